# baseline (speedup 1.0000x reference)
import functools

import jax
import jax.numpy as jnp
from jax import lax
from jax.experimental import pallas as pl
from jax.experimental.pallas import tpu as pltpu
from jax.experimental.pallas import tpu_sc as plsc

B, S, H = 1, 2048, 1024
F = 2048
E = 8
T = B * S
TILE = 256
NW = 24
NWP = 24
ROWS = (NW + 1) * TILE
NEG = -1e30


def _tri(n, strict, dtype=jnp.float32):
    r = lax.broadcasted_iota(jnp.int32, (n, n), 0)
    c = lax.broadcasted_iota(jnp.int32, (n, n), 1)
    return (r > c if strict else r >= c).astype(dtype)


def _router_body(x_ref, wr_ref, posb_ref, gatesb_ref, work_ref):
    xt = x_ref[...]
    logits = jnp.dot(xt, wr_ref[...], preferred_element_type=jnp.float32)
    iota_e = lax.broadcasted_iota(jnp.int32, (T, E), 1)
    m0 = jnp.max(logits, axis=-1, keepdims=True)
    i0 = jnp.min(jnp.where(logits == m0, iota_e, E), axis=-1, keepdims=True)
    masked = jnp.where(iota_e == i0, NEG, logits)
    m1 = jnp.max(masked, axis=-1, keepdims=True)
    i1 = jnp.min(jnp.where(masked == m1, iota_e, E), axis=-1, keepdims=True)
    g0 = 1.0 / (1.0 + jnp.exp(m1 - m0))
    g1 = 1.0 / (1.0 + jnp.exp(m0 - m1))

    sel = ((iota_e == i0) | (iota_e == i1)).astype(jnp.float32)
    lt = _tri(128, strict=True)
    within = []
    bsums = []
    for b in range(T // 128):
        blk = sel[b * 128:(b + 1) * 128]
        within.append(jnp.dot(lt, blk, preferred_element_type=jnp.float32))
        bsums.append(jnp.sum(blk, axis=0, keepdims=True))
    bsum = jnp.concatenate(bsums, axis=0)
    carry = jnp.dot(_tri(16, strict=True), bsum,
                    preferred_element_type=jnp.float32)
    rank = jnp.concatenate(
        [within[b] + carry[b:b + 1] for b in range(T // 128)], axis=0)

    counts = jnp.sum(bsum, axis=0, keepdims=True)
    nt = jnp.floor((counts + (TILE - 1)) / TILE)
    padcnt = nt * TILE
    ut = _tri(E, strict=True).T
    pado = jnp.dot(padcnt, ut, preferred_element_type=jnp.float32)

    pos = pado + rank
    pos0 = jnp.sum(jnp.where(iota_e == i0, pos, 0.0), axis=-1, keepdims=True)
    pos1 = jnp.sum(jnp.where(iota_e == i1, pos, 0.0), axis=-1, keepdims=True)
    posb_ref[...] = (jnp.where(iota_e == 0, pos0, 0.0)
                     + jnp.where(iota_e == 1, pos1, 0.0)).astype(jnp.int32)
    gatesb_ref[...] = (jnp.where(iota_e == 0, g0, 0.0)
                       + jnp.where(iota_e == 1, g1, 0.0))

    cum_items = jnp.dot(nt, _tri(E, strict=False).T,
                        preferred_element_type=jnp.float32)
    base_item = cum_items - nt
    n_items = jnp.max(cum_items)
    wi = lax.broadcasted_iota(jnp.int32, (NWP, E), 0).astype(jnp.float32)
    we = lax.broadcasted_iota(jnp.int32, (NWP, E), 1)
    eid = jnp.sum((wi >= cum_items).astype(jnp.int32), axis=-1,
                  keepdims=True)
    eid = jnp.minimum(eid, E - 1)
    onehot = (we == eid).astype(jnp.float32)
    base_sel = jnp.sum(onehot * base_item, axis=-1, keepdims=True)
    pbase_sel = jnp.sum(onehot * (pado / TILE), axis=-1, keepdims=True)
    w1d = wi[:, :1]
    tid = (pbase_sel + w1d - base_sel).astype(jnp.int32)
    is_pad = w1d >= n_items
    eid = jnp.where(is_pad, E - 1, eid)
    tid = jnp.where(is_pad, NW, tid)
    work_ref[...] = (jnp.where(we == 0, eid, 0)
                     + jnp.where(we == 1, tid, 0)).astype(jnp.int32)


@jax.jit
def _router(xt, Wr):
    return pl.pallas_call(
        _router_body,
        grid=(1,),
        in_specs=[
            pl.BlockSpec((T, H), lambda i: (0, 0)),
            pl.BlockSpec((H, E), lambda i: (0, 0)),
        ],
        out_specs=[
            pl.BlockSpec((T, E), lambda i: (0, 0)),
            pl.BlockSpec((T, E), lambda i: (0, 0)),
            pl.BlockSpec((NWP, E), lambda i: (0, 0)),
        ],
        out_shape=[
            jax.ShapeDtypeStruct((T, E), jnp.int32),
            jax.ShapeDtypeStruct((T, E), jnp.float32),
            jax.ShapeDtypeStruct((NWP, E), jnp.int32),
        ],
    )(xt, Wr)


@functools.lru_cache(maxsize=1)
def _sc_mesh():
    return plsc.VectorSubcoreMesh(core_axis_name="c", subcore_axis_name="s")


_NWORK = 32
_TPW = T // _NWORK


def _dispatch_body(x_hbm, pos0_hbm, pos1_hbm, g0_hbm, g1_hbm, xs_hbm, gs_hbm,
                   idx0_v, idx1_v, g0_v, g1_v, rows_v, sem0, sem1, semg):
    wid = lax.axis_index("s") * 2 + lax.axis_index("c")
    base = wid * _TPW
    pltpu.sync_copy(pos0_hbm.at[pl.ds(base, _TPW)], idx0_v)
    pltpu.sync_copy(pos1_hbm.at[pl.ds(base, _TPW)], idx1_v)
    pltpu.sync_copy(g0_hbm.at[pl.ds(base, _TPW)], g0_v)
    pltpu.sync_copy(g1_hbm.at[pl.ds(base, _TPW)], g1_v)
    pltpu.sync_copy(x_hbm.at[pl.ds(base, _TPW)], rows_v)
    cg0 = pltpu.async_copy(g0_v, gs_hbm.at[idx0_v], semg)
    cg1 = pltpu.async_copy(g1_v, gs_hbm.at[idx1_v], semg)
    c0 = pltpu.async_copy(rows_v, xs_hbm.at[idx0_v], sem0)
    c1 = pltpu.async_copy(rows_v, xs_hbm.at[idx1_v], sem1)
    cg0.wait()
    cg1.wait()
    c0.wait()
    c1.wait()


@jax.jit
def _dispatch(xt, pos0, pos1, g0, g1):
    return pl.kernel(
        _dispatch_body,
        out_type=(jax.ShapeDtypeStruct((ROWS, H), jnp.float32),
                  jax.ShapeDtypeStruct((ROWS,), jnp.float32)),
        mesh=_sc_mesh(),
        scratch_types=[
            pltpu.VMEM((_TPW,), jnp.int32),
            pltpu.VMEM((_TPW,), jnp.int32),
            pltpu.VMEM((_TPW,), jnp.float32),
            pltpu.VMEM((_TPW,), jnp.float32),
            pltpu.VMEM((_TPW, H), jnp.float32),
            pltpu.SemaphoreType.DMA,
            pltpu.SemaphoreType.DMA,
            pltpu.SemaphoreType.DMA,
        ],
    )(xt, pos0, pos1, g0, g1)


def _gmm_body(s_ref, xs_ref, gs_ref, w1_ref, w2_ref, ys_ref):
    w = pl.program_id(0)

    @pl.when(s_ref[w, 1] != NW)
    def _():
        xb = xs_ref[...]
        h = jnp.dot(xb, w1_ref[0], preferred_element_type=jnp.float32)
        h = jax.nn.gelu(h)
        y = jnp.dot(h, w2_ref[0], preferred_element_type=jnp.float32)
        ys_ref[...] = (gs_ref[...] * y).astype(jnp.bfloat16)


@jax.jit
def _gmm(work, xsb, gs2, W1b, W2b):
    grid_spec = pltpu.PrefetchScalarGridSpec(
        num_scalar_prefetch=1,
        grid=(NW,),
        in_specs=[
            pl.BlockSpec((TILE, H), lambda w, s: (s[w, 1], 0)),
            pl.BlockSpec((TILE, 1), lambda w, s: (s[w, 1], 0)),
            pl.BlockSpec((1, H, F), lambda w, s: (s[w, 0], 0, 0)),
            pl.BlockSpec((1, F, H), lambda w, s: (s[w, 0], 0, 0)),
        ],
        out_specs=pl.BlockSpec((TILE, H), lambda w, s: (s[w, 1], 0)),
    )
    return pl.pallas_call(
        _gmm_body,
        grid_spec=grid_spec,
        out_shape=jax.ShapeDtypeStruct((ROWS, H), jnp.bfloat16),
    )(work, xsb, gs2, W1b, W2b)


def _combine_body(ys_hbm, pos0_hbm, pos1_hbm, y0_hbm, y1_hbm,
                  idx0_v, idx1_v, rows_v, sem):
    wid = lax.axis_index("s") * 2 + lax.axis_index("c")
    base = wid * _TPW
    pltpu.sync_copy(pos0_hbm.at[pl.ds(base, _TPW)], idx0_v)
    pltpu.sync_copy(pos1_hbm.at[pl.ds(base, _TPW)], idx1_v)
    pltpu.async_copy(ys_hbm.at[idx0_v], rows_v, sem).wait()
    pltpu.sync_copy(rows_v, y0_hbm.at[pl.ds(base, _TPW)])
    pltpu.async_copy(ys_hbm.at[idx1_v], rows_v, sem).wait()
    pltpu.sync_copy(rows_v, y1_hbm.at[pl.ds(base, _TPW)])


@jax.jit
def _combine(ysp, pos0, pos1):
    return pl.kernel(
        _combine_body,
        out_type=(jax.ShapeDtypeStruct((T, H // 2), jnp.int32),
                  jax.ShapeDtypeStruct((T, H // 2), jnp.int32)),
        mesh=_sc_mesh(),
        scratch_types=[
            pltpu.VMEM((_TPW,), jnp.int32),
            pltpu.VMEM((_TPW,), jnp.int32),
            pltpu.VMEM((_TPW, H // 2), jnp.int32),
            pltpu.SemaphoreType.DMA,
        ],
    )(ysp, pos0, pos1)


def _add_body(y0_ref, y1_ref, out_ref):
    out_ref[...] = (y0_ref[...].astype(jnp.float32)
                    + y1_ref[...].astype(jnp.float32))


@jax.jit
def _add(y0, y1):
    return pl.pallas_call(
        _add_body,
        grid=(T // 256,),
        in_specs=[
            pl.BlockSpec((256, H), lambda j: (j, 0)),
            pl.BlockSpec((256, H), lambda j: (j, 0)),
        ],
        out_specs=pl.BlockSpec((256, H), lambda j: (j, 0)),
        out_shape=jax.ShapeDtypeStruct((T, H), jnp.float32),
    )(y0, y1)


def kernel(x, Wr, W1, W2):
    xt = x.reshape(T, H)
    posb, gatesb, work = _router(xt, Wr)
    pos0 = posb[:, 0]
    pos1 = posb[:, 1]
    xs, gs = _dispatch(xt, pos0, pos1, gatesb[:, 0], gatesb[:, 1])
    ys = _gmm(work, xs, gs.reshape(ROWS, 1), W1, W2)
    ysp = lax.bitcast_convert_type(ys.reshape(ROWS, H // 2, 2), jnp.int32)
    y0p, y1p = _combine(ysp, pos0, pos1)
    y0 = lax.bitcast_convert_type(y0p[..., None], jnp.bfloat16).reshape(T, H)
    y1 = lax.bitcast_convert_type(y1p[..., None], jnp.bfloat16).reshape(T, H)
    out = _add(y0, y1)
    return out.reshape(B, S, H)


# speedup vs baseline: 2.4653x; 2.4653x over previous
import functools

import jax
import jax.numpy as jnp
from jax import lax
from jax.experimental import pallas as pl
from jax.experimental.pallas import tpu as pltpu
from jax.experimental.pallas import tpu_sc as plsc

B, S, H = 1, 2048, 1024
F = 2048
E = 8
T = B * S
TILE = 256
NW = 24
NWP = 24
ROWS = (NW + 1) * TILE
NEG = -1e30


def _tri(n, strict, dtype=jnp.float32):
    r = lax.broadcasted_iota(jnp.int32, (n, n), 0)
    c = lax.broadcasted_iota(jnp.int32, (n, n), 1)
    return (r > c if strict else r >= c).astype(dtype)


def _router_body(x_ref, wr_ref, posb_ref, gatesb_ref, work_ref):
    xt = x_ref[...]
    logits = jnp.dot(xt, wr_ref[...], preferred_element_type=jnp.float32)
    iota_e = lax.broadcasted_iota(jnp.int32, (T, E), 1)
    m0 = jnp.max(logits, axis=-1, keepdims=True)
    i0 = jnp.min(jnp.where(logits == m0, iota_e, E), axis=-1, keepdims=True)
    masked = jnp.where(iota_e == i0, NEG, logits)
    m1 = jnp.max(masked, axis=-1, keepdims=True)
    i1 = jnp.min(jnp.where(masked == m1, iota_e, E), axis=-1, keepdims=True)
    g0 = 1.0 / (1.0 + jnp.exp(m1 - m0))
    g1 = 1.0 / (1.0 + jnp.exp(m0 - m1))

    sel = ((iota_e == i0) | (iota_e == i1)).astype(jnp.float32)
    lt = _tri(128, strict=True)
    within = []
    bsums = []
    for b in range(T // 128):
        blk = sel[b * 128:(b + 1) * 128]
        within.append(jnp.dot(lt, blk, preferred_element_type=jnp.float32))
        bsums.append(jnp.sum(blk, axis=0, keepdims=True))
    bsum = jnp.concatenate(bsums, axis=0)
    carry = jnp.dot(_tri(16, strict=True), bsum,
                    preferred_element_type=jnp.float32)
    rank = jnp.concatenate(
        [within[b] + carry[b:b + 1] for b in range(T // 128)], axis=0)

    counts = jnp.sum(bsum, axis=0, keepdims=True)
    nt = jnp.floor((counts + (TILE - 1)) / TILE)
    padcnt = nt * TILE
    ut = _tri(E, strict=True).T
    pado = jnp.dot(padcnt, ut, preferred_element_type=jnp.float32)

    pos = pado + rank
    pos0 = jnp.sum(jnp.where(iota_e == i0, pos, 0.0), axis=-1, keepdims=True)
    pos1 = jnp.sum(jnp.where(iota_e == i1, pos, 0.0), axis=-1, keepdims=True)
    posb_ref[...] = (jnp.where(iota_e == 0, pos0, 0.0)
                     + jnp.where(iota_e == 1, pos1, 0.0)).astype(jnp.int32)
    gatesb_ref[...] = (jnp.where(iota_e == 0, g0, 0.0)
                       + jnp.where(iota_e == 1, g1, 0.0))

    cum_items = jnp.dot(nt, _tri(E, strict=False).T,
                        preferred_element_type=jnp.float32)
    base_item = cum_items - nt
    n_items = jnp.max(cum_items)
    wi = lax.broadcasted_iota(jnp.int32, (NWP, E), 0).astype(jnp.float32)
    we = lax.broadcasted_iota(jnp.int32, (NWP, E), 1)
    eid = jnp.sum((wi >= cum_items).astype(jnp.int32), axis=-1,
                  keepdims=True)
    eid = jnp.minimum(eid, E - 1)
    onehot = (we == eid).astype(jnp.float32)
    base_sel = jnp.sum(onehot * base_item, axis=-1, keepdims=True)
    pbase_sel = jnp.sum(onehot * (pado / TILE), axis=-1, keepdims=True)
    w1d = wi[:, :1]
    tid = (pbase_sel + w1d - base_sel).astype(jnp.int32)
    is_pad = w1d >= n_items
    first = ((w1d == base_sel) & ~is_pad).astype(jnp.int32)
    nonempty = (nt > 0.0)
    korder = jnp.sum((nonempty & (we < eid)).astype(jnp.int32),
                     axis=-1, keepdims=True)
    nxt = jnp.min(jnp.where(nonempty & (we > eid), we, E),
                  axis=-1, keepdims=True)
    nxt = jnp.where(nxt >= E, eid, nxt)
    eid = jnp.where(is_pad, E - 1, eid)
    tid = jnp.where(is_pad, NW, tid)
    work_ref[...] = (jnp.where(we == 0, eid, 0)
                     + jnp.where(we == 1, tid, 0)
                     + jnp.where(we == 2, first, 0)
                     + jnp.where(we == 3, nxt, 0)
                     + jnp.where(we == 4, korder, 0)).astype(jnp.int32)


@jax.jit
def _router(xt, Wr):
    return pl.pallas_call(
        _router_body,
        grid=(1,),
        in_specs=[
            pl.BlockSpec((T, H), lambda i: (0, 0)),
            pl.BlockSpec((H, E), lambda i: (0, 0)),
        ],
        out_specs=[
            pl.BlockSpec((T, E), lambda i: (0, 0)),
            pl.BlockSpec((T, E), lambda i: (0, 0)),
            pl.BlockSpec((NWP, E), lambda i: (0, 0)),
        ],
        out_shape=[
            jax.ShapeDtypeStruct((T, E), jnp.int32),
            jax.ShapeDtypeStruct((T, E), jnp.float32),
            jax.ShapeDtypeStruct((NWP, E), jnp.int32),
        ],
    )(xt, Wr)


@functools.lru_cache(maxsize=1)
def _sc_mesh():
    return plsc.VectorSubcoreMesh(core_axis_name="c", subcore_axis_name="s")


_NWORK = 32
_TPW = T // _NWORK


def _dispatch_body(x_hbm, pos0_hbm, pos1_hbm, g0_hbm, g1_hbm, xs_hbm, gs_hbm,
                   idx0_v, idx1_v, g0_v, g1_v, rows_v, sem0, sem1, semg):
    wid = lax.axis_index("s") * 2 + lax.axis_index("c")
    base = wid * _TPW
    pltpu.sync_copy(pos0_hbm.at[pl.ds(base, _TPW)], idx0_v)
    pltpu.sync_copy(pos1_hbm.at[pl.ds(base, _TPW)], idx1_v)
    pltpu.sync_copy(g0_hbm.at[pl.ds(base, _TPW)], g0_v)
    pltpu.sync_copy(g1_hbm.at[pl.ds(base, _TPW)], g1_v)
    pltpu.sync_copy(x_hbm.at[pl.ds(base, _TPW)], rows_v)
    cg0 = pltpu.async_copy(g0_v, gs_hbm.at[idx0_v], semg)
    cg1 = pltpu.async_copy(g1_v, gs_hbm.at[idx1_v], semg)
    c0 = pltpu.async_copy(rows_v, xs_hbm.at[idx0_v], sem0)
    c1 = pltpu.async_copy(rows_v, xs_hbm.at[idx1_v], sem1)
    cg0.wait()
    cg1.wait()
    c0.wait()
    c1.wait()


@jax.jit
def _dispatch(xt, pos0, pos1, g0, g1):
    return pl.kernel(
        _dispatch_body,
        out_type=(jax.ShapeDtypeStruct((ROWS, H), jnp.float32),
                  jax.ShapeDtypeStruct((ROWS,), jnp.float32)),
        mesh=_sc_mesh(),
        scratch_types=[
            pltpu.VMEM((_TPW,), jnp.int32),
            pltpu.VMEM((_TPW,), jnp.int32),
            pltpu.VMEM((_TPW,), jnp.float32),
            pltpu.VMEM((_TPW,), jnp.float32),
            pltpu.VMEM((_TPW, H), jnp.float32),
            pltpu.SemaphoreType.DMA,
            pltpu.SemaphoreType.DMA,
            pltpu.SemaphoreType.DMA,
        ],
    )(xt, pos0, pos1, g0, g1)


def _gmm_body(s_ref, xs_ref, gs_ref, w1_hbm, w2_hbm, ys_ref,
              w1b, w2b, sem1, sem2):
    w = pl.program_id(0)
    eid = s_ref[w, 0]
    tid = s_ref[w, 1]
    first = s_ref[w, 2]
    nxt = s_ref[w, 3]
    k = s_ref[w, 4]
    slot = lax.rem(k, 2)

    @pl.when(w == 0)
    def _():
        pltpu.async_copy(w1_hbm.at[eid], w1b.at[slot], sem1)
        pltpu.async_copy(w2_hbm.at[eid], w2b.at[slot], sem2)

    @pl.when(first == 1)
    def _():
        pltpu.make_async_copy(w1_hbm.at[eid], w1b.at[slot], sem1).wait()
        pltpu.make_async_copy(w2_hbm.at[eid], w2b.at[slot], sem2).wait()

        @pl.when(nxt != eid)
        def _():
            nslot = lax.rem(k + 1, 2)
            pltpu.async_copy(w1_hbm.at[nxt], w1b.at[nslot], sem1)
            pltpu.async_copy(w2_hbm.at[nxt], w2b.at[nslot], sem2)

    def compute(w1r, w2r):
        xb = xs_ref[...]
        h = jnp.dot(xb, w1r, preferred_element_type=jnp.float32)
        h = jax.nn.gelu(h)
        y = jnp.dot(h, w2r, preferred_element_type=jnp.float32)
        ys_ref[...] = gs_ref[...] * y

    @pl.when((tid != NW) & (slot == 0))
    def _():
        compute(w1b[0], w2b[0])

    @pl.when((tid != NW) & (slot == 1))
    def _():
        compute(w1b[1], w2b[1])


@jax.jit
def _gmm(work, xsb, gs2, W1b, W2b):
    grid_spec = pltpu.PrefetchScalarGridSpec(
        num_scalar_prefetch=1,
        grid=(NW,),
        in_specs=[
            pl.BlockSpec((TILE, H), lambda w, s: (s[w, 1], 0)),
            pl.BlockSpec((TILE, 1), lambda w, s: (s[w, 1], 0)),
            pl.BlockSpec(memory_space=pl.ANY),
            pl.BlockSpec(memory_space=pl.ANY),
        ],
        out_specs=pl.BlockSpec((TILE, H), lambda w, s: (s[w, 1], 0)),
        scratch_shapes=[
            pltpu.VMEM((2, H, F), jnp.float32),
            pltpu.VMEM((2, F, H), jnp.float32),
            pltpu.SemaphoreType.DMA,
            pltpu.SemaphoreType.DMA,
        ],
    )
    return pl.pallas_call(
        _gmm_body,
        grid_spec=grid_spec,
        out_shape=jax.ShapeDtypeStruct((ROWS, H), jnp.float32),
    )(work, xsb, gs2, W1b, W2b)


def _combine_body(ys_hbm, pos0_hbm, pos1_hbm, y0_hbm, y1_hbm,
                  idx0_v, idx1_v, rows_v, sem):
    wid = lax.axis_index("s") * 2 + lax.axis_index("c")
    base = wid * _TPW
    pltpu.sync_copy(pos0_hbm.at[pl.ds(base, _TPW)], idx0_v)
    pltpu.sync_copy(pos1_hbm.at[pl.ds(base, _TPW)], idx1_v)
    pltpu.async_copy(ys_hbm.at[idx0_v], rows_v, sem).wait()
    pltpu.sync_copy(rows_v, y0_hbm.at[pl.ds(base, _TPW)])
    pltpu.async_copy(ys_hbm.at[idx1_v], rows_v, sem).wait()
    pltpu.sync_copy(rows_v, y1_hbm.at[pl.ds(base, _TPW)])


@jax.jit
def _combine(ys, pos0, pos1):
    return pl.kernel(
        _combine_body,
        out_type=(jax.ShapeDtypeStruct((T, H), jnp.float32),
                  jax.ShapeDtypeStruct((T, H), jnp.float32)),
        mesh=_sc_mesh(),
        scratch_types=[
            pltpu.VMEM((_TPW,), jnp.int32),
            pltpu.VMEM((_TPW,), jnp.int32),
            pltpu.VMEM((_TPW, H), jnp.float32),
            pltpu.SemaphoreType.DMA,
        ],
    )(ys, pos0, pos1)


def _add_body(y0_ref, y1_ref, out_ref):
    out_ref[...] = y0_ref[...] + y1_ref[...]


@jax.jit
def _add(y0, y1):
    return pl.pallas_call(
        _add_body,
        grid=(T // 256,),
        in_specs=[
            pl.BlockSpec((256, H), lambda j: (j, 0)),
            pl.BlockSpec((256, H), lambda j: (j, 0)),
        ],
        out_specs=pl.BlockSpec((256, H), lambda j: (j, 0)),
        out_shape=jax.ShapeDtypeStruct((T, H), jnp.float32),
    )(y0, y1)


def kernel(x, Wr, W1, W2):
    xt = x.reshape(T, H)
    posb, gatesb, work = _router(xt, Wr)
    pos0 = posb[:, 0]
    pos1 = posb[:, 1]
    xs, gs = _dispatch(xt, pos0, pos1, gatesb[:, 0], gatesb[:, 1])
    ys = _gmm(work, xs, gs.reshape(ROWS, 1), W1, W2)
    y0, y1 = _combine(ys, pos0, pos1)
    out = _add(y0, y1)
    return out.reshape(B, S, H)


# speedup vs baseline: 3.0369x; 1.2319x over previous
import functools

import jax
import jax.numpy as jnp
from jax import lax
from jax.experimental import pallas as pl
from jax.experimental.pallas import tpu as pltpu
from jax.experimental.pallas import tpu_sc as plsc

B, S, H = 1, 2048, 1024
F = 2048
E = 8
T = B * S
TILE = 256
NW = 24
NWP = 24
ROWS = (NW + 1) * TILE
NEG = -1e30


def _tri(n, strict, dtype=jnp.float32):
    r = lax.broadcasted_iota(jnp.int32, (n, n), 0)
    c = lax.broadcasted_iota(jnp.int32, (n, n), 1)
    return (r > c if strict else r >= c).astype(dtype)


def _router_body(x_ref, wr_ref, posb_ref, gatesb_ref, work_ref):
    xt = x_ref[...]
    logits = jnp.dot(xt, wr_ref[...], preferred_element_type=jnp.float32)
    iota_e = lax.broadcasted_iota(jnp.int32, (T, E), 1)
    m0 = jnp.max(logits, axis=-1, keepdims=True)
    i0 = jnp.min(jnp.where(logits == m0, iota_e, E), axis=-1, keepdims=True)
    masked = jnp.where(iota_e == i0, NEG, logits)
    m1 = jnp.max(masked, axis=-1, keepdims=True)
    i1 = jnp.min(jnp.where(masked == m1, iota_e, E), axis=-1, keepdims=True)
    g0 = 1.0 / (1.0 + jnp.exp(m1 - m0))
    g1 = 1.0 / (1.0 + jnp.exp(m0 - m1))

    sel = ((iota_e == i0) | (iota_e == i1)).astype(jnp.float32)
    lt = _tri(128, strict=True)
    within = []
    bsums = []
    for b in range(T // 128):
        blk = sel[b * 128:(b + 1) * 128]
        within.append(jnp.dot(lt, blk, preferred_element_type=jnp.float32))
        bsums.append(jnp.sum(blk, axis=0, keepdims=True))
    bsum = jnp.concatenate(bsums, axis=0)
    carry = jnp.dot(_tri(16, strict=True), bsum,
                    preferred_element_type=jnp.float32)
    rank = jnp.concatenate(
        [within[b] + carry[b:b + 1] for b in range(T // 128)], axis=0)

    counts = jnp.sum(bsum, axis=0, keepdims=True)
    nt = jnp.floor((counts + (TILE - 1)) / TILE)
    padcnt = nt * TILE
    ut = _tri(E, strict=True).T
    pado = jnp.dot(padcnt, ut, preferred_element_type=jnp.float32)

    pos = pado + rank
    pos0 = jnp.sum(jnp.where(iota_e == i0, pos, 0.0), axis=-1, keepdims=True)
    pos1 = jnp.sum(jnp.where(iota_e == i1, pos, 0.0), axis=-1, keepdims=True)
    posb_ref[...] = (jnp.where(iota_e == 0, pos0, 0.0)
                     + jnp.where(iota_e == 1, pos1, 0.0)).astype(jnp.int32)
    gatesb_ref[...] = (jnp.where(iota_e == 0, g0, 0.0)
                       + jnp.where(iota_e == 1, g1, 0.0))

    cum_items = jnp.dot(nt, _tri(E, strict=False).T,
                        preferred_element_type=jnp.float32)
    base_item = cum_items - nt
    n_items = jnp.max(cum_items)
    wi = lax.broadcasted_iota(jnp.int32, (NWP, E), 0).astype(jnp.float32)
    we = lax.broadcasted_iota(jnp.int32, (NWP, E), 1)
    eid = jnp.sum((wi >= cum_items).astype(jnp.int32), axis=-1,
                  keepdims=True)
    eid = jnp.minimum(eid, E - 1)
    onehot = (we == eid).astype(jnp.float32)
    base_sel = jnp.sum(onehot * base_item, axis=-1, keepdims=True)
    pbase_sel = jnp.sum(onehot * (pado / TILE), axis=-1, keepdims=True)
    w1d = wi[:, :1]
    tid = (pbase_sel + w1d - base_sel).astype(jnp.int32)
    is_pad = w1d >= n_items
    first = ((w1d == base_sel) & ~is_pad).astype(jnp.int32)
    nonempty = (nt > 0.0)
    korder = jnp.sum((nonempty & (we < eid)).astype(jnp.int32),
                     axis=-1, keepdims=True)
    nxt = jnp.min(jnp.where(nonempty & (we > eid), we, E),
                  axis=-1, keepdims=True)
    nxt = jnp.where(nxt >= E, eid, nxt)
    eid = jnp.where(is_pad, E - 1, eid)
    tid = jnp.where(is_pad, NW, tid)
    work_ref[...] = (jnp.where(we == 0, eid, 0)
                     + jnp.where(we == 1, tid, 0)
                     + jnp.where(we == 2, first, 0)
                     + jnp.where(we == 3, nxt, 0)
                     + jnp.where(we == 4, korder, 0)).astype(jnp.int32)


@jax.jit
def _router(xt, Wr):
    return pl.pallas_call(
        _router_body,
        grid=(1,),
        in_specs=[
            pl.BlockSpec((T, H), lambda i: (0, 0)),
            pl.BlockSpec((H, E), lambda i: (0, 0)),
        ],
        out_specs=[
            pl.BlockSpec((T, E), lambda i: (0, 0)),
            pl.BlockSpec((T, E), lambda i: (0, 0)),
            pl.BlockSpec((NWP, E), lambda i: (0, 0)),
        ],
        out_shape=[
            jax.ShapeDtypeStruct((T, E), jnp.int32),
            jax.ShapeDtypeStruct((T, E), jnp.float32),
            jax.ShapeDtypeStruct((NWP, E), jnp.int32),
        ],
    )(xt, Wr)


@functools.lru_cache(maxsize=1)
def _sc_mesh():
    return plsc.VectorSubcoreMesh(core_axis_name="c", subcore_axis_name="s")


_NWORK = 32
_TPW = T // _NWORK


def _dispatch_body(x_hbm, pos0_hbm, pos1_hbm, xs_hbm,
                   idx0_v, idx1_v, rows_v, sem0, sem1):
    wid = lax.axis_index("s") * 2 + lax.axis_index("c")
    base = wid * _TPW
    pltpu.sync_copy(pos0_hbm.at[pl.ds(base, _TPW)], idx0_v)
    pltpu.sync_copy(pos1_hbm.at[pl.ds(base, _TPW)], idx1_v)
    pltpu.sync_copy(x_hbm.at[pl.ds(base, _TPW)], rows_v)
    c0 = pltpu.async_copy(rows_v, xs_hbm.at[idx0_v], sem0)
    c1 = pltpu.async_copy(rows_v, xs_hbm.at[idx1_v], sem1)
    c0.wait()
    c1.wait()


@jax.jit
def _dispatch(xt, pos0, pos1):
    return pl.kernel(
        _dispatch_body,
        out_type=jax.ShapeDtypeStruct((ROWS, H), jnp.float32),
        mesh=_sc_mesh(),
        scratch_types=[
            pltpu.VMEM((_TPW,), jnp.int32),
            pltpu.VMEM((_TPW,), jnp.int32),
            pltpu.VMEM((_TPW, H), jnp.float32),
            pltpu.SemaphoreType.DMA,
            pltpu.SemaphoreType.DMA,
        ],
    )(xt, pos0, pos1)


def _gmm_body(s_ref, xs_ref, w1_hbm, w2_hbm, ys_ref,
              w1b, w2b, sem1, sem2):
    w = pl.program_id(0)
    eid = s_ref[w, 0]
    tid = s_ref[w, 1]
    first = s_ref[w, 2]
    nxt = s_ref[w, 3]
    k = s_ref[w, 4]
    slot = lax.rem(k, 2)

    @pl.when(w == 0)
    def _():
        pltpu.async_copy(w1_hbm.at[eid], w1b.at[slot], sem1)
        pltpu.async_copy(w2_hbm.at[eid], w2b.at[slot], sem2)

    @pl.when(first == 1)
    def _():
        pltpu.make_async_copy(w1_hbm.at[eid], w1b.at[slot], sem1).wait()
        pltpu.make_async_copy(w2_hbm.at[eid], w2b.at[slot], sem2).wait()

        @pl.when(nxt != eid)
        def _():
            nslot = lax.rem(k + 1, 2)
            pltpu.async_copy(w1_hbm.at[nxt], w1b.at[nslot], sem1)
            pltpu.async_copy(w2_hbm.at[nxt], w2b.at[nslot], sem2)

    def compute(w1r, w2r):
        xb = xs_ref[...]
        h = jnp.dot(xb, w1r, preferred_element_type=jnp.float32)
        h = jax.nn.gelu(h)
        ys_ref[...] = jnp.dot(h, w2r, preferred_element_type=jnp.float32)

    @pl.when((tid != NW) & (slot == 0))
    def _():
        compute(w1b[0], w2b[0])

    @pl.when((tid != NW) & (slot == 1))
    def _():
        compute(w1b[1], w2b[1])


@jax.jit
def _gmm(work, xsb, W1b, W2b):
    grid_spec = pltpu.PrefetchScalarGridSpec(
        num_scalar_prefetch=1,
        grid=(NW,),
        in_specs=[
            pl.BlockSpec((TILE, H), lambda w, s: (s[w, 1], 0)),
            pl.BlockSpec(memory_space=pl.ANY),
            pl.BlockSpec(memory_space=pl.ANY),
        ],
        out_specs=pl.BlockSpec((TILE, H), lambda w, s: (s[w, 1], 0)),
        scratch_shapes=[
            pltpu.VMEM((2, H, F), jnp.float32),
            pltpu.VMEM((2, F, H), jnp.float32),
            pltpu.SemaphoreType.DMA,
            pltpu.SemaphoreType.DMA,
        ],
    )
    return pl.pallas_call(
        _gmm_body,
        grid_spec=grid_spec,
        out_shape=jax.ShapeDtypeStruct((ROWS, H), jnp.float32),
    )(work, xsb, W1b, W2b)


def _combine_body(ys_hbm, pos0_hbm, pos1_hbm, y0_hbm, y1_hbm,
                  idx0_v, idx1_v, rows_v, sem):
    wid = lax.axis_index("s") * 2 + lax.axis_index("c")
    base = wid * _TPW
    pltpu.sync_copy(pos0_hbm.at[pl.ds(base, _TPW)], idx0_v)
    pltpu.sync_copy(pos1_hbm.at[pl.ds(base, _TPW)], idx1_v)
    pltpu.async_copy(ys_hbm.at[idx0_v], rows_v, sem).wait()
    pltpu.sync_copy(rows_v, y0_hbm.at[pl.ds(base, _TPW)])
    pltpu.async_copy(ys_hbm.at[idx1_v], rows_v, sem).wait()
    pltpu.sync_copy(rows_v, y1_hbm.at[pl.ds(base, _TPW)])


@jax.jit
def _combine(ys, pos0, pos1):
    return pl.kernel(
        _combine_body,
        out_type=(jax.ShapeDtypeStruct((T, H), jnp.float32),
                  jax.ShapeDtypeStruct((T, H), jnp.float32)),
        mesh=_sc_mesh(),
        scratch_types=[
            pltpu.VMEM((_TPW,), jnp.int32),
            pltpu.VMEM((_TPW,), jnp.int32),
            pltpu.VMEM((_TPW, H), jnp.float32),
            pltpu.SemaphoreType.DMA,
        ],
    )(ys, pos0, pos1)


def _fma_body(g_ref, y0_ref, y1_ref, out_ref):
    g = g_ref[...]
    out_ref[...] = g[:, 0:1] * y0_ref[...] + g[:, 1:2] * y1_ref[...]


@jax.jit
def _fma(gatesb, y0, y1):
    return pl.pallas_call(
        _fma_body,
        grid=(T // 256,),
        in_specs=[
            pl.BlockSpec((256, E), lambda j: (j, 0)),
            pl.BlockSpec((256, H), lambda j: (j, 0)),
            pl.BlockSpec((256, H), lambda j: (j, 0)),
        ],
        out_specs=pl.BlockSpec((256, H), lambda j: (j, 0)),
        out_shape=jax.ShapeDtypeStruct((T, H), jnp.float32),
    )(gatesb, y0, y1)


def kernel(x, Wr, W1, W2):
    xt = x.reshape(T, H)
    posb, gatesb, work = _router(xt, Wr)
    pos0 = posb[:, 0]
    pos1 = posb[:, 1]
    xs = _dispatch(xt, pos0, pos1)
    ys = _gmm(work, xs, W1, W2)
    y0, y1 = _combine(ys, pos0, pos1)
    out = _fma(gatesb, y0, y1)
    return out.reshape(B, S, H)
